# baseline (speedup 1.0000x reference)
import functools

import jax
import jax.numpy as jnp
from jax import lax
from jax.experimental import pallas as pl
from jax.experimental.pallas import tpu as pltpu
from jax.experimental.pallas import tpu_sc as plsc

B = 16384
L = 50
D = 50
DP = 56
LANES = 16

_info = plsc.get_sparse_core_info()
NC, NS = _info.num_cores, _info.num_subcores
NW = NC * NS
SPW = B // NW
C = 16
NCHUNK = SPW // C
RPC = C * L

_mesh = plsc.VectorSubcoreMesh(core_axis_name="c", subcore_axis_name="s")


@functools.partial(
    pl.kernel,
    out_type=jax.ShapeDtypeStruct((B, DP), jnp.float32),
    mesh=_mesh,
    scratch_types=[
        pltpu.VMEM((SPW, L), jnp.int32),
        pltpu.VMEM((RPC, DP), jnp.float32),
        pltpu.VMEM((RPC, DP), jnp.float32),
        pltpu.VMEM((C, DP), jnp.float32),
        pltpu.SemaphoreType.DMA,
        pltpu.SemaphoreType.DMA,
    ],
    compiler_params=pltpu.CompilerParams(use_tc_tiling_on_sc=False),
)
def _gram_mean(idx_hbm, table_hbm, out_hbm, idx_v, rows0, rows1, out_v,
               sem0, sem1):
    wid = lax.axis_index("s") * NC + lax.axis_index("c")
    scale = jnp.float32(1.0 / L)

    pltpu.sync_copy(idx_hbm.at[pl.ds(wid * SPW, SPW)], idx_v)

    def fire(c, buf, sem):
        for i in range(C):
            pltpu.async_copy(
                table_hbm.at[idx_v.at[c * C + i]],
                buf.at[pl.ds(i * L, L)],
                sem,
            )

    def drain(buf, sem):
        pltpu.make_async_copy(table_hbm.at[pl.ds(0, RPC)], buf, sem).wait()

    fire(0, rows0, sem0)

    @pl.loop(0, NCHUNK, step=2)
    def _pair(c0):
        for b in range(2):
            c = c0 + b
            cur, csem = (rows0, sem0) if b == 0 else (rows1, sem1)
            nxt, nsem = (rows1, sem1) if b == 0 else (rows0, sem0)

            @pl.when(c + 1 < NCHUNK)
            def _():
                fire(c + 1, nxt, nsem)

            drain(cur, csem)

            @pl.loop(0, C)
            def _sample(s):
                base = s * L
                z = jnp.zeros((LANES,), jnp.float32)
                a0, a1, a2, a3 = z, z, z, z
                for r in range(L):
                    a0 = a0 + cur[base + r, pl.ds(0, LANES)]
                    a1 = a1 + cur[base + r, pl.ds(16, LANES)]
                    a2 = a2 + cur[base + r, pl.ds(32, LANES)]
                    a3 = a3 + cur[base + r, pl.ds(40, LANES)]
                out_v[s, pl.ds(0, LANES)] = a0 * scale
                out_v[s, pl.ds(16, LANES)] = a1 * scale
                out_v[s, pl.ds(32, LANES)] = a2 * scale
                out_v[s, pl.ds(40, LANES)] = a3 * scale

            pltpu.sync_copy(out_v, out_hbm.at[pl.ds(wid * SPW + c * C, C)])


def kernel(icdcode, embed_table):
    idx = icdcode.astype(jnp.int32)
    table_p = jnp.pad(embed_table.astype(jnp.float32), ((0, 0), (0, DP - D)))
    return _gram_mean(idx, table_p)[:, :D]


# speedup vs baseline: 1.1496x; 1.1496x over previous
import functools

import jax
import jax.numpy as jnp
from jax import lax
from jax.experimental import pallas as pl
from jax.experimental.pallas import tpu as pltpu
from jax.experimental.pallas import tpu_sc as plsc

B = 16384
L = 50
D = 50
DT = 64
DP = 56
LANES = 16

_info = plsc.get_sparse_core_info()
NC, NS = _info.num_cores, _info.num_subcores
NW = NC * NS
SPW = B // NW
C = 16
NCHUNK = SPW // C
RPC = C * L
GB = 80
NGB = RPC // GB
NIDX = NCHUNK * NGB

_mesh = plsc.VectorSubcoreMesh(core_axis_name="c", subcore_axis_name="s")


@functools.partial(
    pl.kernel,
    out_type=jax.ShapeDtypeStruct((B, DP), jnp.float32),
    mesh=_mesh,
    scratch_types=[
        pltpu.VMEM((NIDX, GB), jnp.int32),
        pltpu.VMEM((RPC, DT), jnp.bfloat16),
        pltpu.VMEM((RPC, DT), jnp.bfloat16),
        pltpu.VMEM((C, DP), jnp.float32),
        pltpu.SemaphoreType.DMA,
        pltpu.SemaphoreType.DMA,
    ],
    compiler_params=pltpu.CompilerParams(
        use_tc_tiling_on_sc=False, needs_layout_passes=False
    ),
)
def _gram_mean(idx_hbm, table_hbm, out_hbm, idx_v, rows0, rows1, out_v,
               sem0, sem1):
    wid = lax.axis_index("s") * NC + lax.axis_index("c")
    scale = jnp.float32(1.0 / L)

    pltpu.sync_copy(idx_hbm.at[pl.ds(wid * NIDX, NIDX)], idx_v)

    def fire(c, buf, sem):
        for k in range(NGB):
            pltpu.async_copy(
                table_hbm.at[idx_v.at[c * NGB + k]],
                buf.at[pl.ds(k * GB, GB)],
                sem,
            )

    def drain(buf, sem):
        pltpu.make_async_copy(table_hbm.at[pl.ds(0, RPC)], buf, sem).wait()

    fire(0, rows0, sem0)

    iota = lax.iota(jnp.int32, LANES)
    col_e0 = 2 * iota
    col_o0 = col_e0 + 1
    col_e1 = col_e0 + 32
    col_o1 = col_o0 + 32
    msk1 = col_e1 < DP
    msk1o = col_o1 < DP

    @pl.loop(0, NCHUNK, step=2)
    def _pair(c0):
        for b in range(2):
            c = c0 + b
            cur, csem = (rows0, sem0) if b == 0 else (rows1, sem1)
            nxt, nsem = (rows1, sem1) if b == 0 else (rows0, sem0)

            @pl.when(c + 1 < NCHUNK)
            def _():
                fire(c + 1, nxt, nsem)

            drain(cur, csem)

            @pl.loop(0, C)
            def _sample(s):
                base = s * L
                z = jnp.zeros((LANES,), jnp.float32)
                e0 = o0 = e1 = o1 = z
                for r in range(L):
                    v0 = cur[base + r, pl.ds(0, 2 * LANES)]
                    v1 = cur[base + r, pl.ds(32, 2 * LANES)]
                    a, bb = plsc.unpack(v0, format=plsc.PackFormat.INTERLEAVED)
                    d, e = plsc.unpack(v1, format=plsc.PackFormat.INTERLEAVED)
                    e0 = e0 + a
                    o0 = o0 + bb
                    e1 = e1 + d
                    o1 = o1 + e
                row = jnp.full((LANES,), s, jnp.int32)
                plsc.store_scatter(out_v, [row, col_e0], e0 * scale)
                plsc.store_scatter(out_v, [row, col_o0], o0 * scale)
                plsc.store_scatter(out_v, [row, col_e1], e1 * scale, mask=msk1)
                plsc.store_scatter(out_v, [row, col_o1], o1 * scale, mask=msk1o)

            pltpu.sync_copy(out_v, out_hbm.at[pl.ds(wid * SPW + c * C, C)])


def kernel(icdcode, embed_table):
    idx_2d = icdcode.reshape(B * L // GB, GB).astype(jnp.int32)
    table_p = jnp.pad(embed_table.astype(jnp.bfloat16), ((0, 0), (0, DT - D)))
    return _gram_mean(idx_2d, table_p)[:, :D]
